# baseline (speedup 1.0000x reference)
import functools

import jax
import jax.numpy as jnp
from jax import lax
from jax.experimental import pallas as pl
from jax.experimental.pallas import tpu as pltpu
from jax.experimental.pallas import tpu_sc as plsc

B = 16384
F = 26
K = 32
N_WORKERS = 32
CHUNK = B // N_WORKERS
SUB = 32
NSUB = CHUNK // SUB
L = 16

_mesh = plsc.VectorSubcoreMesh(core_axis_name="c", subcore_axis_name="s")

_GDN = lax.GatherDimensionNumbers(
    offset_dims=(), collapsed_slice_dims=(0,), start_index_map=(0,))


def _perm(x, idx):
    return lax.gather(x, idx[:, None], _GDN, slice_sizes=(1,),
                      mode=lax.GatherScatterMode.PROMISE_IN_BOUNDS)


def _lane_sum(x, lane):
    for d in (1, 2, 4, 8):
        x = x + _perm(x, lane ^ d)
    return x


N = 1000000
TW = 512
NGRP = N // TW
NTAIL = N - NGRP * TW
GPT = (NGRP + N_WORKERS - 1) // N_WORKERS


@functools.partial(
    pl.kernel,
    mesh=_mesh,
    compiler_params=pltpu.CompilerParams(needs_layout_passes=False),
    out_type=jax.ShapeDtypeStruct((N * K,), jnp.float32),
    scratch_types=[
        pltpu.VMEM((8, TW), jnp.float32),
        pltpu.VMEM((8, TW), jnp.float32),
        pltpu.VMEM((8, TW), jnp.float32),
        pltpu.VMEM((8, TW), jnp.float32),
        pltpu.VMEM((8, TW), jnp.float32),
        pltpu.VMEM((8, TW), jnp.float32),
        pltpu.VMEM((8, TW), jnp.float32),
        pltpu.VMEM((8, TW), jnp.float32),
        pltpu.VMEM((TW * K,), jnp.float32),
        pltpu.VMEM((TW * K,), jnp.float32),
        pltpu.SemaphoreType.DMA,
        pltpu.SemaphoreType.DMA,
        pltpu.SemaphoreType.DMA,
        pltpu.SemaphoreType.DMA,
    ],
)
def _transpose_sc(vt_hbm, vtail_hbm, out_hbm,
                  b00, b01, b02, b03, b10, b11, b12, b13,
                  o0, o1, si0, si1, so0, so1):
    w = lax.axis_index("s") * 2 + lax.axis_index("c")
    bands = ((b00, b01, b02, b03), (b10, b11, b12, b13))
    outs = (o0, o1)
    sis = (si0, si1)
    sos = (so0, so1)
    lane = lax.broadcasted_iota(jnp.int32, (L,), 0)
    lane_k = lane * K

    def start_in(g, buf):
        c0 = g * TW
        for a in range(4):
            pltpu.make_async_copy(
                vt_hbm.at[pl.ds(8 * a, 8), pl.ds(c0, TW)],
                bands[buf][a], sis[buf]).start()

    def wait_in(buf):
        for a in range(4):
            pltpu.make_async_copy(
                vt_hbm.at[pl.ds(0, 8), pl.ds(0, TW)],
                bands[buf][a], sis[buf]).wait()

    def out_copy(g, buf):
        return pltpu.make_async_copy(
            outs[buf], out_hbm.at[pl.ds(g * (TW * K), TW * K)], sos[buf])

    def transpose_group(buf, width):
        def jbody(jj, _):
            base = lane_k + jj * (L * K)
            for a in range(4):
                for r in range(8):
                    v = bands[buf][a][r, pl.ds(jj * L, L)]
                    plsc.store_scatter(outs[buf], [base + (8 * a + r)], v)
            return 0
        lax.fori_loop(0, width // L, jbody, 0)

    start_in(w, 0)

    @pl.when(N_WORKERS + w < NGRP)
    def _():
        start_in(N_WORKERS + w, 1)

    def pair_body(t, _):
        for b in (0, 1):
            i = 2 * t + b
            g = i * N_WORKERS + w

            @pl.when(g < NGRP)
            def _():
                wait_in(b)

                @pl.when(i >= 2)
                def _():
                    out_copy(g, b).wait()

                transpose_group(b, TW)
                out_copy(g, b).start()
                g2 = (i + 2) * N_WORKERS + w

                @pl.when(g2 < NGRP)
                def _():
                    start_in(g2, b)
        return 0

    lax.fori_loop(0, GPT // 2, pair_body, 0)

    out_copy(0, 0).wait()
    out_copy(0, 1).wait()

    @pl.when(w == 0)
    def _():
        pltpu.sync_copy(vtail_hbm, outs[0].at[pl.ds(0, NTAIL * K)])
        pltpu.sync_copy(
            outs[0].at[pl.ds(0, NTAIL * K)],
            out_hbm.at[pl.ds(NGRP * TW * K, NTAIL * K)])


@functools.partial(
    pl.kernel,
    mesh=_mesh,
    compiler_params=pltpu.CompilerParams(use_tc_tiling_on_sc=False),
    out_type=jax.ShapeDtypeStruct((B,), jnp.float32),
    scratch_types=[
        pltpu.VMEM((CHUNK * F,), jnp.int32),
        pltpu.VMEM((CHUNK * F,), jnp.float32),
        pltpu.VMEM((2, SUB * F, K), jnp.float32),
        pltpu.VMEM((SUB,), jnp.float32),
        pltpu.VMEM((L,), jnp.float32),
        pltpu.SemaphoreType.DMA,
        pltpu.SemaphoreType.DMA,
        pltpu.SemaphoreType.DMA,
    ],
)
def _fm_sc(x_hbm, lw_hbm, vw_hbm, b_hbm, out_hbm,
           idx_v, lw_v, rows_v, out_v, bias_v, sem0, sem1, sem_lw):
    wid = lax.axis_index("s") * 2 + lax.axis_index("c")
    base = wid * CHUNK
    sems = (sem0, sem1)

    pltpu.sync_copy(x_hbm.at[pl.ds(base * F, CHUNK * F)], idx_v)
    pltpu.sync_copy(b_hbm, bias_v)
    bias_vec = bias_v[...]
    lane = lax.broadcasted_iota(jnp.int32, (L,), 0)

    def gather_rows(j, buf):
        return pltpu.make_async_copy(
            vw_hbm.at[idx_v.at[pl.ds(j * (SUB * F), SUB * F)]],
            rows_v.at[buf],
            sems[buf],
        )

    lw_copy = pltpu.make_async_copy(lw_hbm.at[idx_v], lw_v, sem_lw)
    lw_copy.start()
    gather_rows(0, 0).start()
    lw_copy.wait()

    def compute_sub(j, buf):
        rows = rows_v.at[buf]
        for g in range(SUB // L):
            def s_body(s16, out_vec):
                s = g * L + s16
                rbase = s * F
                lbase = j * (SUB * F) + s * F
                a0 = [None] * 4
                a1 = [None] * 4
                sq = [None] * 8
                for f in range(F):
                    r0 = rows[rbase + f, pl.ds(0, L)]
                    r1 = rows[rbase + f, pl.ds(L, L)]
                    p = f % 4
                    a0[p] = r0 if a0[p] is None else a0[p] + r0
                    a1[p] = r1 if a1[p] is None else a1[p] + r1
                    m0 = r0 * r0
                    m1 = r1 * r1
                    sq[p] = m0 if sq[p] is None else sq[p] + m0
                    sq[p + 4] = m1 if sq[p + 4] is None else sq[p + 4] + m1
                s0 = (a0[0] + a0[1]) + (a0[2] + a0[3])
                s1 = (a1[0] + a1[1]) + (a1[2] + a1[3])
                qq = ((sq[0] + sq[1]) + (sq[2] + sq[3])) + (
                    (sq[4] + sq[5]) + (sq[6] + sq[7]))
                l0 = lw_v[pl.ds(lbase, L)]
                l1 = lw_v[pl.ds(lbase + L, L)]
                linv = l0 + jnp.where(lane < (F - L), l1, 0.0)
                tot = 0.5 * (s0 * s0 + s1 * s1 - qq) + linv
                tot = _lane_sum(tot, lane)
                return jnp.where(lane == s16, tot, out_vec)

            out_vec = lax.fori_loop(0, L, s_body, jnp.zeros((L,), jnp.float32))
            out_v[pl.ds(g * L, L)] = out_vec + bias_vec

        pltpu.sync_copy(out_v, out_hbm.at[pl.ds(base + j * SUB, SUB)])

    def pair_body(t, _):
        for b in (0, 1):
            j = 2 * t + b
            nxt = j + 1

            @pl.when(nxt < NSUB)
            def _():
                gather_rows(nxt, 1 - b).start()

            gather_rows(j, b).wait()
            compute_sub(j, b)
        return 0

    lax.fori_loop(0, NSUB // 2, pair_body, 0)


def kernel(x, linear_w, v_w, b):
    xf = x.reshape(-1).astype(jnp.int32)
    lwf = linear_w.reshape(-1)
    b16 = jnp.broadcast_to(b.astype(jnp.float32), (L,))
    vtail = v_w[NGRP * TW:].reshape(-1)
    tlin = _transpose_sc(v_w.T, vtail)
    vw2 = jnp.reshape(tlin, (N, K))
    out = _fm_sc(xf, lwf, vw2, b16)
    return out.reshape(B, 1)


# speedup vs baseline: 1.2634x; 1.2634x over previous
import functools

import jax
import jax.numpy as jnp
from jax import lax
from jax.experimental import pallas as pl
from jax.experimental.pallas import tpu as pltpu
from jax.experimental.pallas import tpu_sc as plsc

B = 16384
F = 26
K = 32
N_WORKERS = 32
CHUNK = B // N_WORKERS
SUB = 32
NSUB = CHUNK // SUB
L = 16

_mesh = plsc.VectorSubcoreMesh(core_axis_name="c", subcore_axis_name="s")

_GDN = lax.GatherDimensionNumbers(
    offset_dims=(), collapsed_slice_dims=(0,), start_index_map=(0,))


def _perm(x, idx):
    return lax.gather(x, idx[:, None], _GDN, slice_sizes=(1,),
                      mode=lax.GatherScatterMode.PROMISE_IN_BOUNDS)


def _lane_sum(x, lane):
    for d in (1, 2, 4, 8):
        x = x + _perm(x, lane ^ d)
    return x


N = 1000000
TW = 512
NGRP = N // TW
NTAIL = N - NGRP * TW
GPT = (NGRP + N_WORKERS - 1) // N_WORKERS


@functools.partial(
    pl.kernel,
    mesh=_mesh,
    compiler_params=pltpu.CompilerParams(needs_layout_passes=False),
    out_type=jax.ShapeDtypeStruct((N * K,), jnp.float32),
    scratch_types=[
        pltpu.VMEM((8, TW), jnp.float32),
        pltpu.VMEM((8, TW), jnp.float32),
        pltpu.VMEM((8, TW), jnp.float32),
        pltpu.VMEM((8, TW), jnp.float32),
        pltpu.VMEM((8, TW), jnp.float32),
        pltpu.VMEM((8, TW), jnp.float32),
        pltpu.VMEM((8, TW), jnp.float32),
        pltpu.VMEM((8, TW), jnp.float32),
        pltpu.VMEM((TW * K,), jnp.float32),
        pltpu.VMEM((TW * K,), jnp.float32),
        pltpu.SemaphoreType.DMA,
        pltpu.SemaphoreType.DMA,
        pltpu.SemaphoreType.DMA,
        pltpu.SemaphoreType.DMA,
    ],
)
def _transpose_sc(vt_hbm, vtail_hbm, out_hbm,
                  b00, b01, b02, b03, b10, b11, b12, b13,
                  o0, o1, si0, si1, so0, so1):
    w = lax.axis_index("s") * 2 + lax.axis_index("c")
    bands = ((b00, b01, b02, b03), (b10, b11, b12, b13))
    outs = (o0, o1)
    sis = (si0, si1)
    sos = (so0, so1)
    lane = lax.broadcasted_iota(jnp.int32, (L,), 0)
    lane_k = lane * K

    def start_in(g, buf):
        c0 = g * TW
        for a in range(4):
            pltpu.make_async_copy(
                vt_hbm.at[pl.ds(8 * a, 8), pl.ds(c0, TW)],
                bands[buf][a], sis[buf]).start()

    def wait_in(buf):
        for a in range(4):
            pltpu.make_async_copy(
                vt_hbm.at[pl.ds(0, 8), pl.ds(0, TW)],
                bands[buf][a], sis[buf]).wait()

    def out_copy(g, buf):
        return pltpu.make_async_copy(
            outs[buf], out_hbm.at[pl.ds(g * (TW * K), TW * K)], sos[buf])

    full = lane >= 0

    def transpose_group(buf, width):
        def jbody(jj, _):
            base = lane_k + jj * (L * K)
            vs = [bands[buf][a][r, pl.ds(jj * L, L)]
                  for a in range(4) for r in range(8)]
            for kk, v in enumerate(vs):
                plsc.store_scatter(outs[buf], [base + kk], v, mask=full)
            return 0
        lax.fori_loop(0, width // L, jbody, 0)

    start_in(w, 0)

    @pl.when(N_WORKERS + w < NGRP)
    def _():
        start_in(N_WORKERS + w, 1)

    def pair_body(t, _):
        for b in (0, 1):
            i = 2 * t + b
            g = i * N_WORKERS + w

            @pl.when(g < NGRP)
            def _():
                wait_in(b)

                @pl.when(i >= 2)
                def _():
                    out_copy(g, b).wait()

                transpose_group(b, TW)
                out_copy(g, b).start()
                g2 = (i + 2) * N_WORKERS + w

                @pl.when(g2 < NGRP)
                def _():
                    start_in(g2, b)
        return 0

    lax.fori_loop(0, GPT // 2, pair_body, 0)

    out_copy(0, 0).wait()
    out_copy(0, 1).wait()

    @pl.when(w == 0)
    def _():
        pltpu.sync_copy(vtail_hbm, outs[0].at[pl.ds(0, NTAIL * K)])
        pltpu.sync_copy(
            outs[0].at[pl.ds(0, NTAIL * K)],
            out_hbm.at[pl.ds(NGRP * TW * K, NTAIL * K)])


@functools.partial(
    pl.kernel,
    mesh=_mesh,
    compiler_params=pltpu.CompilerParams(use_tc_tiling_on_sc=False),
    out_type=jax.ShapeDtypeStruct((B,), jnp.float32),
    scratch_types=[
        pltpu.VMEM((CHUNK * F,), jnp.int32),
        pltpu.VMEM((CHUNK * F,), jnp.float32),
        pltpu.VMEM((2, SUB * F, K), jnp.float32),
        pltpu.VMEM((SUB,), jnp.float32),
        pltpu.VMEM((L,), jnp.float32),
        pltpu.SemaphoreType.DMA,
        pltpu.SemaphoreType.DMA,
        pltpu.SemaphoreType.DMA,
    ],
)
def _fm_sc(x_hbm, lw_hbm, vw_hbm, b_hbm, out_hbm,
           idx_v, lw_v, rows_v, out_v, bias_v, sem0, sem1, sem_lw):
    wid = lax.axis_index("s") * 2 + lax.axis_index("c")
    base = wid * CHUNK
    sems = (sem0, sem1)

    pltpu.sync_copy(x_hbm.at[pl.ds(base * F, CHUNK * F)], idx_v)
    pltpu.sync_copy(b_hbm, bias_v)
    bias_vec = bias_v[...]
    lane = lax.broadcasted_iota(jnp.int32, (L,), 0)

    def gather_rows(j, buf):
        return pltpu.make_async_copy(
            vw_hbm.at[idx_v.at[pl.ds(j * (SUB * F), SUB * F)]],
            rows_v.at[buf],
            sems[buf],
        )

    lw_copy = pltpu.make_async_copy(lw_hbm.at[idx_v], lw_v, sem_lw)
    lw_copy.start()
    gather_rows(0, 0).start()
    lw_copy.wait()

    def compute_sub(j, buf):
        rows = rows_v.at[buf]
        for g in range(SUB // L):
            def s_body(s16, out_vec):
                s = g * L + s16
                rbase = s * F
                lbase = j * (SUB * F) + s * F
                a0 = [None] * 4
                a1 = [None] * 4
                sq = [None] * 8
                for f in range(F):
                    r0 = rows[rbase + f, pl.ds(0, L)]
                    r1 = rows[rbase + f, pl.ds(L, L)]
                    p = f % 4
                    a0[p] = r0 if a0[p] is None else a0[p] + r0
                    a1[p] = r1 if a1[p] is None else a1[p] + r1
                    m0 = r0 * r0
                    m1 = r1 * r1
                    sq[p] = m0 if sq[p] is None else sq[p] + m0
                    sq[p + 4] = m1 if sq[p + 4] is None else sq[p + 4] + m1
                s0 = (a0[0] + a0[1]) + (a0[2] + a0[3])
                s1 = (a1[0] + a1[1]) + (a1[2] + a1[3])
                qq = ((sq[0] + sq[1]) + (sq[2] + sq[3])) + (
                    (sq[4] + sq[5]) + (sq[6] + sq[7]))
                l0 = lw_v[pl.ds(lbase, L)]
                l1 = lw_v[pl.ds(lbase + L, L)]
                linv = l0 + jnp.where(lane < (F - L), l1, 0.0)
                tot = 0.5 * (s0 * s0 + s1 * s1 - qq) + linv
                tot = _lane_sum(tot, lane)
                return jnp.where(lane == s16, tot, out_vec)

            out_vec = lax.fori_loop(0, L, s_body, jnp.zeros((L,), jnp.float32))
            out_v[pl.ds(g * L, L)] = out_vec + bias_vec

        pltpu.sync_copy(out_v, out_hbm.at[pl.ds(base + j * SUB, SUB)])

    def pair_body(t, _):
        for b in (0, 1):
            j = 2 * t + b
            nxt = j + 1

            @pl.when(nxt < NSUB)
            def _():
                gather_rows(nxt, 1 - b).start()

            gather_rows(j, b).wait()
            compute_sub(j, b)
        return 0

    lax.fori_loop(0, NSUB // 2, pair_body, 0)


def kernel(x, linear_w, v_w, b):
    xf = x.reshape(-1).astype(jnp.int32)
    lwf = linear_w.reshape(-1)
    b16 = jnp.broadcast_to(b.astype(jnp.float32), (L,))
    vtail = v_w[NGRP * TW:].reshape(-1)
    tlin = _transpose_sc(v_w.T, vtail)
    vw2 = jnp.reshape(tlin, (N, K))
    out = _fm_sc(xf, lwf, vw2, b16)
    return out.reshape(B, 1)


# speedup vs baseline: 1.2737x; 1.0081x over previous
import functools

import jax
import jax.numpy as jnp
from jax import lax
from jax.experimental import pallas as pl
from jax.experimental.pallas import tpu as pltpu
from jax.experimental.pallas import tpu_sc as plsc

B = 16384
F = 26
K = 32
N_WORKERS = 32
CHUNK = B // N_WORKERS
SUB = 32
NSUB = CHUNK // SUB
L = 16

_mesh = plsc.VectorSubcoreMesh(core_axis_name="c", subcore_axis_name="s")

_GDN = lax.GatherDimensionNumbers(
    offset_dims=(), collapsed_slice_dims=(0,), start_index_map=(0,))


def _perm(x, idx):
    return lax.gather(x, idx[:, None], _GDN, slice_sizes=(1,),
                      mode=lax.GatherScatterMode.PROMISE_IN_BOUNDS)


def _lane_sum(x, lane):
    for d in (1, 2, 4, 8):
        x = x + _perm(x, lane ^ d)
    return x


N = 1000000
TW = 768
NGRP = N // TW
NTAIL = N - NGRP * TW
GPT = (NGRP + N_WORKERS - 1) // N_WORKERS


@functools.partial(
    pl.kernel,
    mesh=_mesh,
    compiler_params=pltpu.CompilerParams(needs_layout_passes=False),
    out_type=jax.ShapeDtypeStruct((N * K,), jnp.float32),
    scratch_types=[
        pltpu.VMEM((K, TW), jnp.float32),
        pltpu.VMEM((K, TW), jnp.float32),
        pltpu.VMEM((TW * K,), jnp.float32),
        pltpu.VMEM((TW * K,), jnp.float32),
        pltpu.SemaphoreType.DMA,
        pltpu.SemaphoreType.DMA,
        pltpu.SemaphoreType.DMA,
        pltpu.SemaphoreType.DMA,
    ],
)
def _transpose_sc(vt_hbm, vtail_hbm, out_hbm,
                  bb0, bb1, o0, o1, si0, si1, so0, so1):
    w = lax.axis_index("s") * 2 + lax.axis_index("c")
    bands = (bb0, bb1)
    outs = (o0, o1)
    sis = (si0, si1)
    sos = (so0, so1)
    lane = lax.broadcasted_iota(jnp.int32, (L,), 0)
    lane_k = lane * K

    def in_copy(g, buf):
        c0 = g * TW
        return pltpu.make_async_copy(
            vt_hbm.at[pl.ds(0, K), pl.ds(c0, TW)], bands[buf], sis[buf])

    def start_in(g, buf):
        in_copy(g, buf).start()

    def wait_in(buf):
        in_copy(0, buf).wait()

    def out_copy(g, buf):
        return pltpu.make_async_copy(
            outs[buf], out_hbm.at[pl.ds(g * (TW * K), TW * K)], sos[buf])

    full = lane >= 0

    def transpose_group(buf, width):
        def jbody(jj, _):
            base = lane_k + jj * (L * K)
            vs = [bands[buf][kk, pl.ds(jj * L, L)] for kk in range(K)]
            for kk, v in enumerate(vs):
                plsc.store_scatter(outs[buf], [base + kk], v, mask=full)
            return 0
        lax.fori_loop(0, width // L, jbody, 0)

    start_in(w, 0)

    @pl.when(N_WORKERS + w < NGRP)
    def _():
        start_in(N_WORKERS + w, 1)

    def pair_body(t, _):
        for b in (0, 1):
            i = 2 * t + b
            g = i * N_WORKERS + w

            @pl.when(g < NGRP)
            def _():
                wait_in(b)

                @pl.when(i >= 2)
                def _():
                    out_copy(g, b).wait()

                transpose_group(b, TW)
                out_copy(g, b).start()
                g2 = (i + 2) * N_WORKERS + w

                @pl.when(g2 < NGRP)
                def _():
                    start_in(g2, b)
        return 0

    lax.fori_loop(0, (GPT + 1) // 2, pair_body, 0)

    out_copy(0, 0).wait()
    out_copy(0, 1).wait()

    @pl.when(w == 0)
    def _():
        pltpu.sync_copy(vtail_hbm, outs[0].at[pl.ds(0, NTAIL * K)])
        pltpu.sync_copy(
            outs[0].at[pl.ds(0, NTAIL * K)],
            out_hbm.at[pl.ds(NGRP * TW * K, NTAIL * K)])


@functools.partial(
    pl.kernel,
    mesh=_mesh,
    compiler_params=pltpu.CompilerParams(use_tc_tiling_on_sc=False),
    out_type=jax.ShapeDtypeStruct((B,), jnp.float32),
    scratch_types=[
        pltpu.VMEM((CHUNK * F,), jnp.int32),
        pltpu.VMEM((CHUNK * F,), jnp.float32),
        pltpu.VMEM((2, SUB * F, K), jnp.float32),
        pltpu.VMEM((SUB,), jnp.float32),
        pltpu.VMEM((L,), jnp.float32),
        pltpu.SemaphoreType.DMA,
        pltpu.SemaphoreType.DMA,
        pltpu.SemaphoreType.DMA,
    ],
)
def _fm_sc(x_hbm, lw_hbm, vw_hbm, b_hbm, out_hbm,
           idx_v, lw_v, rows_v, out_v, bias_v, sem0, sem1, sem_lw):
    wid = lax.axis_index("s") * 2 + lax.axis_index("c")
    base = wid * CHUNK
    sems = (sem0, sem1)

    pltpu.sync_copy(x_hbm.at[pl.ds(base * F, CHUNK * F)], idx_v)
    pltpu.sync_copy(b_hbm, bias_v)
    bias_vec = bias_v[...]
    lane = lax.broadcasted_iota(jnp.int32, (L,), 0)

    def gather_rows(j, buf):
        return pltpu.make_async_copy(
            vw_hbm.at[idx_v.at[pl.ds(j * (SUB * F), SUB * F)]],
            rows_v.at[buf],
            sems[buf],
        )

    lw_copy = pltpu.make_async_copy(lw_hbm.at[idx_v], lw_v, sem_lw)
    lw_copy.start()
    gather_rows(0, 0).start()
    lw_copy.wait()

    def compute_sub(j, buf):
        rows = rows_v.at[buf]
        for g in range(SUB // L):
            def s_body(s16, out_vec):
                s = g * L + s16
                rbase = s * F
                lbase = j * (SUB * F) + s * F
                a0 = [None] * 4
                a1 = [None] * 4
                sq = [None] * 8
                for f in range(F):
                    r0 = rows[rbase + f, pl.ds(0, L)]
                    r1 = rows[rbase + f, pl.ds(L, L)]
                    p = f % 4
                    a0[p] = r0 if a0[p] is None else a0[p] + r0
                    a1[p] = r1 if a1[p] is None else a1[p] + r1
                    m0 = r0 * r0
                    m1 = r1 * r1
                    sq[p] = m0 if sq[p] is None else sq[p] + m0
                    sq[p + 4] = m1 if sq[p + 4] is None else sq[p + 4] + m1
                s0 = (a0[0] + a0[1]) + (a0[2] + a0[3])
                s1 = (a1[0] + a1[1]) + (a1[2] + a1[3])
                qq = ((sq[0] + sq[1]) + (sq[2] + sq[3])) + (
                    (sq[4] + sq[5]) + (sq[6] + sq[7]))
                l0 = lw_v[pl.ds(lbase, L)]
                l1 = lw_v[pl.ds(lbase + L, L)]
                linv = l0 + jnp.where(lane < (F - L), l1, 0.0)
                tot = 0.5 * (s0 * s0 + s1 * s1 - qq) + linv
                tot = _lane_sum(tot, lane)
                return jnp.where(lane == s16, tot, out_vec)

            out_vec = lax.fori_loop(0, L, s_body, jnp.zeros((L,), jnp.float32))
            out_v[pl.ds(g * L, L)] = out_vec + bias_vec

        pltpu.sync_copy(out_v, out_hbm.at[pl.ds(base + j * SUB, SUB)])

    def pair_body(t, _):
        for b in (0, 1):
            j = 2 * t + b
            nxt = j + 1

            @pl.when(nxt < NSUB)
            def _():
                gather_rows(nxt, 1 - b).start()

            gather_rows(j, b).wait()
            compute_sub(j, b)
        return 0

    lax.fori_loop(0, NSUB // 2, pair_body, 0)


def kernel(x, linear_w, v_w, b):
    xf = x.reshape(-1).astype(jnp.int32)
    lwf = linear_w.reshape(-1)
    b16 = jnp.broadcast_to(b.astype(jnp.float32), (L,))
    vtail = v_w[NGRP * TW:].reshape(-1)
    tlin = _transpose_sc(v_w.T, vtail)
    vw2 = jnp.reshape(tlin, (N, K))
    out = _fm_sc(xf, lwf, vw2, b16)
    return out.reshape(B, 1)


# speedup vs baseline: 1.3405x; 1.0525x over previous
import functools

import jax
import jax.numpy as jnp
from jax import lax
from jax.experimental import pallas as pl
from jax.experimental.pallas import tpu as pltpu
from jax.experimental.pallas import tpu_sc as plsc

B = 16384
F = 26
K = 32
N_WORKERS = 32
CHUNK = B // N_WORKERS
SUB = 32
NSUB = CHUNK // SUB
L = 16

_mesh = plsc.VectorSubcoreMesh(core_axis_name="c", subcore_axis_name="s")

_GDN = lax.GatherDimensionNumbers(
    offset_dims=(), collapsed_slice_dims=(0,), start_index_map=(0,))


def _perm(x, idx):
    return lax.gather(x, idx[:, None], _GDN, slice_sizes=(1,),
                      mode=lax.GatherScatterMode.PROMISE_IN_BOUNDS)


def _lane_sum(x, lane):
    for d in (1, 2, 4, 8):
        x = x + _perm(x, lane ^ d)
    return x


N = 1000000
TW = 768
NGRP = N // TW
NTAIL = N - NGRP * TW
GPT = (NGRP + N_WORKERS - 1) // N_WORKERS


@functools.partial(
    pl.kernel,
    mesh=_mesh,
    compiler_params=pltpu.CompilerParams(needs_layout_passes=False),
    out_type=jax.ShapeDtypeStruct((N * K,), jnp.float32),
    scratch_types=[
        pltpu.VMEM((K, TW), jnp.float32),
        pltpu.VMEM((K, TW), jnp.float32),
        pltpu.VMEM((TW * K,), jnp.float32),
        pltpu.VMEM((TW * K,), jnp.float32),
        pltpu.SemaphoreType.DMA,
        pltpu.SemaphoreType.DMA,
        pltpu.SemaphoreType.DMA,
        pltpu.SemaphoreType.DMA,
    ],
)
def _transpose_sc(vt_hbm, vtail_hbm, out_hbm,
                  bb0, bb1, o0, o1, si0, si1, so0, so1):
    w = lax.axis_index("s") * 2 + lax.axis_index("c")
    bands = (bb0, bb1)
    outs = (o0, o1)
    sis = (si0, si1)
    sos = (so0, so1)
    lane = lax.broadcasted_iota(jnp.int32, (L,), 0)
    lane_k = lane * K

    def in_copy(g, buf):
        c0 = g * TW
        return pltpu.make_async_copy(
            vt_hbm.at[pl.ds(0, K), pl.ds(c0, TW)], bands[buf], sis[buf])

    def start_in(g, buf):
        in_copy(g, buf).start()

    def wait_in(buf):
        in_copy(0, buf).wait()

    def out_copy(g, buf):
        return pltpu.make_async_copy(
            outs[buf], out_hbm.at[pl.ds(g * (TW * K), TW * K)], sos[buf])

    full = lane >= 0

    def transpose_group(buf, width):
        def jbody(jj, _):
            base = lane_k + jj * (L * K)
            vs = {}
            for kk in range(K + 4):
                if kk < K:
                    vs[kk] = bands[buf][kk, pl.ds(jj * L, L)]
                if kk >= 4:
                    plsc.store_scatter(
                        outs[buf], [base + (kk - 4)], vs.pop(kk - 4), mask=full)
            return 0
        lax.fori_loop(0, width // L, jbody, 0)

    start_in(w, 0)

    @pl.when(N_WORKERS + w < NGRP)
    def _():
        start_in(N_WORKERS + w, 1)

    def pair_body(t, _):
        for b in (0, 1):
            i = 2 * t + b
            g = i * N_WORKERS + w

            @pl.when(g < NGRP)
            def _():
                wait_in(b)

                @pl.when(i >= 2)
                def _():
                    out_copy(g, b).wait()

                transpose_group(b, TW)
                out_copy(g, b).start()
                g2 = (i + 2) * N_WORKERS + w

                @pl.when(g2 < NGRP)
                def _():
                    start_in(g2, b)
        return 0

    lax.fori_loop(0, (GPT + 1) // 2, pair_body, 0)

    out_copy(0, 0).wait()
    out_copy(0, 1).wait()

    @pl.when(w == 0)
    def _():
        pltpu.sync_copy(vtail_hbm, outs[0].at[pl.ds(0, NTAIL * K)])
        pltpu.sync_copy(
            outs[0].at[pl.ds(0, NTAIL * K)],
            out_hbm.at[pl.ds(NGRP * TW * K, NTAIL * K)])


@functools.partial(
    pl.kernel,
    mesh=_mesh,
    compiler_params=pltpu.CompilerParams(use_tc_tiling_on_sc=False),
    out_type=jax.ShapeDtypeStruct((B,), jnp.float32),
    scratch_types=[
        pltpu.VMEM((CHUNK * F,), jnp.int32),
        pltpu.VMEM((CHUNK * F,), jnp.float32),
        pltpu.VMEM((2, SUB * F, K), jnp.float32),
        pltpu.VMEM((SUB,), jnp.float32),
        pltpu.VMEM((L,), jnp.float32),
        pltpu.SemaphoreType.DMA,
        pltpu.SemaphoreType.DMA,
        pltpu.SemaphoreType.DMA,
    ],
)
def _fm_sc(x_hbm, lw_hbm, vw_hbm, b_hbm, out_hbm,
           idx_v, lw_v, rows_v, out_v, bias_v, sem0, sem1, sem_lw):
    wid = lax.axis_index("s") * 2 + lax.axis_index("c")
    base = wid * CHUNK
    sems = (sem0, sem1)

    pltpu.sync_copy(x_hbm.at[pl.ds(base * F, CHUNK * F)], idx_v)
    pltpu.sync_copy(b_hbm, bias_v)
    bias_vec = bias_v[...]
    lane = lax.broadcasted_iota(jnp.int32, (L,), 0)

    def gather_rows(j, buf):
        return pltpu.make_async_copy(
            vw_hbm.at[idx_v.at[pl.ds(j * (SUB * F), SUB * F)]],
            rows_v.at[buf],
            sems[buf],
        )

    lw_copy = pltpu.make_async_copy(lw_hbm.at[idx_v], lw_v, sem_lw)
    lw_copy.start()
    gather_rows(0, 0).start()
    lw_copy.wait()

    def compute_sub(j, buf):
        rows = rows_v.at[buf]
        for g in range(SUB // L):
            def s_body(s16, out_vec):
                s = g * L + s16
                rbase = s * F
                lbase = j * (SUB * F) + s * F
                a0 = [None] * 4
                a1 = [None] * 4
                sq = [None] * 8
                for f in range(F):
                    r0 = rows[rbase + f, pl.ds(0, L)]
                    r1 = rows[rbase + f, pl.ds(L, L)]
                    p = f % 4
                    a0[p] = r0 if a0[p] is None else a0[p] + r0
                    a1[p] = r1 if a1[p] is None else a1[p] + r1
                    m0 = r0 * r0
                    m1 = r1 * r1
                    sq[p] = m0 if sq[p] is None else sq[p] + m0
                    sq[p + 4] = m1 if sq[p + 4] is None else sq[p + 4] + m1
                s0 = (a0[0] + a0[1]) + (a0[2] + a0[3])
                s1 = (a1[0] + a1[1]) + (a1[2] + a1[3])
                qq = ((sq[0] + sq[1]) + (sq[2] + sq[3])) + (
                    (sq[4] + sq[5]) + (sq[6] + sq[7]))
                l0 = lw_v[pl.ds(lbase, L)]
                l1 = lw_v[pl.ds(lbase + L, L)]
                linv = l0 + jnp.where(lane < (F - L), l1, 0.0)
                tot = 0.5 * (s0 * s0 + s1 * s1 - qq) + linv
                tot = _lane_sum(tot, lane)
                return jnp.where(lane == s16, tot, out_vec)

            out_vec = lax.fori_loop(0, L, s_body, jnp.zeros((L,), jnp.float32))
            out_v[pl.ds(g * L, L)] = out_vec + bias_vec

        pltpu.sync_copy(out_v, out_hbm.at[pl.ds(base + j * SUB, SUB)])

    def pair_body(t, _):
        for b in (0, 1):
            j = 2 * t + b
            nxt = j + 1

            @pl.when(nxt < NSUB)
            def _():
                gather_rows(nxt, 1 - b).start()

            gather_rows(j, b).wait()
            compute_sub(j, b)
        return 0

    lax.fori_loop(0, NSUB // 2, pair_body, 0)


def kernel(x, linear_w, v_w, b):
    xf = x.reshape(-1).astype(jnp.int32)
    lwf = linear_w.reshape(-1)
    b16 = jnp.broadcast_to(b.astype(jnp.float32), (L,))
    vtail = v_w[NGRP * TW:].reshape(-1)
    tlin = _transpose_sc(v_w.T, vtail)
    vw2 = jnp.reshape(tlin, (N, K))
    out = _fm_sc(xf, lwf, vw2, b16)
    return out.reshape(B, 1)
